# Initial kernel scaffold; baseline (speedup 1.0000x reference)
import jax
import jax.numpy as jnp
from jax.experimental import pallas as pl


def kernel(x, edge_attr, params, edge_index, batch):
    raise NotImplementedError("write your pallas kernel here")


# speedup vs baseline: 3.4075x; 3.4075x over previous
import functools

import jax
import jax.numpy as jnp
from jax import lax
from jax.experimental import pallas as pl
from jax.experimental.pallas import tpu as pltpu
from jax.experimental.pallas import tpu_sc as plsc

_N = 10000
_E = 160000
_DIN = 128
_D = 16
_B = 64
_PASSES = 4
_STEPS = 5

_NC = 2
_NS = 16
_NW = _NC * _NS
_EW = _E // _NW
_CH = 128
_NCHUNK = 40
_EWP = _CH * _NCHUNK
_EP = _NW * _EWP
_NRS = 632
_NACC = _NRS * _NS

_F32 = jnp.float32
def _dot(a, b):
    return jnp.dot(a, b, preferred_element_type=_F32)


def _bn_pack(g, beta, m, v):
    return jnp.stack([m, jnp.sqrt(v + 1e-5), g, beta])


def _bn_apply(y, bn):
    return (y - bn[0:1, :]) / bn[1:2, :] * bn[2:3, :] + bn[3:4, :]


def _pad_edges(a):
    a = a.reshape((_NW, _EW) + a.shape[1:])
    pad = [(0, 0), (0, _EWP - _EW)] + [(0, 0)] * (a.ndim - 2)
    return jnp.pad(a, pad).reshape((_EP,) + a.shape[2:])


def _sc_mesh():
    return plsc.VectorSubcoreMesh(core_axis_name="c", subcore_axis_name="s",
                                  num_cores=_NC, num_subcores=_NS)


_SC_PARAMS = pltpu.CompilerParams(use_tc_tiling_on_sc=False)


@jax.jit
def _sc_gather(h, srcp):
    @functools.partial(
        pl.kernel,
        out_type=jax.ShapeDtypeStruct((_EP, _D), _F32),
        mesh=_sc_mesh(),
        compiler_params=_SC_PARAMS,
        scratch_types=[
            pltpu.VMEM((_NCHUNK, _CH), jnp.int32),
            pltpu.VMEM((_EWP, _D), _F32),
            pltpu.SemaphoreType.DMA,
        ],
    )
    def k(h_hbm, src_hbm, out_hbm, idx_v, rows_v, sem):
        w = lax.axis_index("s") * _NC + lax.axis_index("c")
        base = w * _EWP
        pltpu.sync_copy(src_hbm.at[w], idx_v)

        def fire(j, _):
            pltpu.async_copy(h_hbm.at[idx_v.at[j]],
                             rows_v.at[pl.ds(j * _CH, _CH)], sem)
            return 0

        lax.fori_loop(0, _NCHUNK, fire, 0)

        def drain(j, _):
            pltpu.make_async_copy(h_hbm.at[idx_v.at[j]],
                                  rows_v.at[pl.ds(j * _CH, _CH)],
                                  sem).wait()
            return 0

        lax.fori_loop(0, _NCHUNK, drain, 0)
        pltpu.sync_copy(rows_v, out_hbm.at[pl.ds(base, _EWP)])

    return k(h, srcp.reshape(_NW, _NCHUNK, _CH))


@jax.jit
def _sc_scatter(msg, dstp):
    @functools.partial(
        pl.kernel,
        out_type=jax.ShapeDtypeStruct((_NC, _NACC, _D), _F32),
        mesh=_sc_mesh(),
        compiler_params=_SC_PARAMS,
        scratch_types=[
            pltpu.VMEM_SHARED((_NACC, _D), _F32),
            pltpu.VMEM((_NRS, _D), _F32),
            pltpu.VMEM((_NCHUNK, _CH), jnp.int32),
            pltpu.VMEM((_EWP, _D), _F32),
            pltpu.SemaphoreType.DMA,
        ],
    )
    def k(msg_hbm, dst_hbm, out_hbm, acc_sh, stage_v, idx_v, val_v, sem):
        c = lax.axis_index("c")
        s = lax.axis_index("s")
        base = (c * _NS + s) * _EWP

        def zrow(i, _):
            stage_v[i, :] = jnp.zeros((_D,), _F32)
            return 0

        lax.fori_loop(0, _NRS, zrow, 0)
        pltpu.sync_copy(stage_v, acc_sh.at[pl.ds(s * _NRS, _NRS)])
        plsc.subcore_barrier()

        pltpu.sync_copy(dst_hbm.at[c * _NS + s], idx_v)
        pltpu.sync_copy(msg_hbm.at[pl.ds(base, _EWP)], val_v)

        def fire(j, _):
            pltpu.async_copy(val_v.at[pl.ds(j * _CH, _CH)],
                             acc_sh.at[idx_v.at[j]], sem, add=True)
            return 0

        lax.fori_loop(0, _NCHUNK, fire, 0)

        def drain(j, _):
            pltpu.make_async_copy(val_v.at[pl.ds(j * _CH, _CH)],
                                  acc_sh.at[idx_v.at[j]], sem).wait()
            return 0

        lax.fori_loop(0, _NCHUNK, drain, 0)
        plsc.subcore_barrier()
        pltpu.sync_copy(acc_sh.at[pl.ds(s * _NRS, _NRS)],
                        out_hbm.at[c, pl.ds(s * _NRS, _NRS)])

    return k(msg, dstp.reshape(_NW, _NCHUNK, _CH))


def _enc_body(x_ref, d0_ref, d1_ref, w_ref, b_ref, bn_ref, h_ref, dg_ref):
    y = _dot(x_ref[...], w_ref[...]) + b_ref[...]
    h_ref[...] = jnp.maximum(_bn_apply(y, bn_ref[...]), 0.0)
    dg_ref[...] = jnp.maximum(d0_ref[...] + d1_ref[...], 1.0)


@jax.jit
def _encoder(x, d0, d1, w0, b0, bn0):
    nb = 5
    blk = _N // nb
    return pl.pallas_call(
        _enc_body,
        grid=(nb,),
        in_specs=[
            pl.BlockSpec((blk, _DIN), lambda i: (i, 0)),
            pl.BlockSpec((blk, _D), lambda i: (i, 0)),
            pl.BlockSpec((blk, _D), lambda i: (i, 0)),
            pl.BlockSpec((_DIN, _D), lambda i: (0, 0)),
            pl.BlockSpec((1, _D), lambda i: (0, 0)),
            pl.BlockSpec((4, _D), lambda i: (0, 0)),
        ],
        out_specs=[
            pl.BlockSpec((blk, _D), lambda i: (i, 0)),
            pl.BlockSpec((blk, _D), lambda i: (i, 0)),
        ],
        out_shape=[
            jax.ShapeDtypeStruct((_N, _D), _F32),
            jax.ShapeDtypeStruct((_N, _D), _F32),
        ],
    )(x, d0, d1, w0, b0, bn0)


def _msg_body(ea_ref, xs_ref, w1_ref, b1_ref, bn1_ref, w2_ref, b2_ref,
              r_ref, f_ref, msg_ref):
    y = _dot(ea_ref[...], w1_ref[...]) + b1_ref[...]
    e2 = jnp.maximum(_bn_apply(y, bn1_ref[...]), 0.0)
    we = _dot(e2, w2_ref[...]) + b2_ref[...]
    xrep = _dot(xs_ref[...], r_ref[...])
    msg_ref[...] = _dot(xrep * we, f_ref[...])


@jax.jit
def _msg(eap, xs, w1, b1, bn1, w2, b2, rmat, fmat):
    be = 2048
    nb = _EP // be
    return pl.pallas_call(
        _msg_body,
        grid=(nb,),
        in_specs=[
            pl.BlockSpec((be, _D), lambda i: (i, 0)),
            pl.BlockSpec((be, _D), lambda i: (i, 0)),
            pl.BlockSpec((_D, 32), lambda i: (0, 0)),
            pl.BlockSpec((1, 32), lambda i: (0, 0)),
            pl.BlockSpec((4, 32), lambda i: (0, 0)),
            pl.BlockSpec((32, 256), lambda i: (0, 0)),
            pl.BlockSpec((1, 256), lambda i: (0, 0)),
            pl.BlockSpec((_D, 256), lambda i: (0, 0)),
            pl.BlockSpec((256, _D), lambda i: (0, 0)),
        ],
        out_specs=pl.BlockSpec((be, _D), lambda i: (i, 0)),
        out_shape=jax.ShapeDtypeStruct((_EP, _D), _F32),
    )(eap, xs, w1, b1, bn1, w2, b2, rmat, fmat)


def _upd_body(a0_ref, a1_ref, dg_ref, h_ref, root_ref, cb_ref, wi_ref,
              wh_ref, bi_ref, bh_ref, out_ref):
    h = h_ref[...]
    agg = (a0_ref[...] + a1_ref[...]) / dg_ref[...]
    conv = agg + _dot(h, root_ref[...]) + cb_ref[...]
    m = jnp.maximum(conv, 0.0)
    gi = _dot(m, wi_ref[...]) + bi_ref[...]
    gh = _dot(h, wh_ref[...]) + bh_ref[...]
    r = jax.nn.sigmoid(gi[:, :_D] + gh[:, :_D])
    z = jax.nn.sigmoid(gi[:, _D:2 * _D] + gh[:, _D:2 * _D])
    n = jnp.tanh(gi[:, 2 * _D:] + r * gh[:, 2 * _D:])
    out_ref[...] = (1.0 - z) * n + z * h


@jax.jit
def _update(a0, a1, deg, h, root, cb, wi, wh, bi, bh):
    nb = 5
    blk = _N // nb
    return pl.pallas_call(
        _upd_body,
        grid=(nb,),
        in_specs=[
            pl.BlockSpec((blk, _D), lambda i: (i, 0)),
            pl.BlockSpec((blk, _D), lambda i: (i, 0)),
            pl.BlockSpec((blk, _D), lambda i: (i, 0)),
            pl.BlockSpec((blk, _D), lambda i: (i, 0)),
            pl.BlockSpec((_D, _D), lambda i: (0, 0)),
            pl.BlockSpec((1, _D), lambda i: (0, 0)),
            pl.BlockSpec((_D, 3 * _D), lambda i: (0, 0)),
            pl.BlockSpec((_D, 3 * _D), lambda i: (0, 0)),
            pl.BlockSpec((1, 3 * _D), lambda i: (0, 0)),
            pl.BlockSpec((1, 3 * _D), lambda i: (0, 0)),
        ],
        out_specs=pl.BlockSpec((blk, _D), lambda i: (i, 0)),
        out_shape=jax.ShapeDtypeStruct((_N, _D), _F32),
    )(a0, a1, deg, h, root, cb, wi, wh, bi, bh)


_SBLK = 2000
_SNB = _N // _SBLK


def _s2s_body(h_ref, b_ref, lwi_ref, lwh_ref, lb_ref, f1_ref, g1_ref,
              bnf1_ref, f2_ref, g2_ref, bnf2_ref, f3_ref, g3_ref, bnf3_ref,
              fo_ref, go_ref, out_ref, ener_ref):
    cols = lax.broadcasted_iota(jnp.int32, (_SBLK, _B), 1)

    def load(i):
        hb = h_ref[pl.ds(i * _SBLK, _SBLK), :]
        bb = b_ref[pl.ds(i * _SBLK, _SBLK), :]
        return hb, (bb == cols).astype(_F32)

    def step(_, carry):
        qh, qc, q_star = carry
        gates = (_dot(q_star, lwi_ref[...]) + lb_ref[0:1, :]
                 + _dot(qh, lwh_ref[...]) + lb_ref[1:2, :])
        g_i = jax.nn.sigmoid(gates[:, :_D])
        g_f = jax.nn.sigmoid(gates[:, _D:2 * _D])
        g_g = jnp.tanh(gates[:, 2 * _D:3 * _D])
        g_o = jax.nn.sigmoid(gates[:, 3 * _D:])
        qc = g_f * qc + g_i * g_g
        qh = g_o * jnp.tanh(qc)
        q = qh

        def sweep1(i, em):
            hb, mask = load(i)
            qb = _dot(mask, q)
            ener = jnp.sum(hb * qb, axis=1, keepdims=True)
            ener_ref[pl.ds(i * _SBLK, _SBLK), :] = ener
            masked = jnp.where(mask > 0.0, ener, -1e30)
            return jnp.maximum(em, jnp.max(masked, axis=0, keepdims=True))

        em = lax.fori_loop(0, _SNB, sweep1, jnp.full((1, _B), -1e30, _F32))

        def sweep2(i, asum):
            _, mask = load(i)
            ener = ener_ref[pl.ds(i * _SBLK, _SBLK), :]
            emaxn = jnp.sum(mask * em, axis=1, keepdims=True)
            a = jnp.exp(ener - emaxn)
            return asum + jnp.sum(mask * a, axis=0, keepdims=True)

        asum = lax.fori_loop(0, _SNB, sweep2, jnp.zeros((1, _B), _F32))

        def sweep3(i, u):
            hb, mask = load(i)
            ener = ener_ref[pl.ds(i * _SBLK, _SBLK), :]
            emaxn = jnp.sum(mask * em, axis=1, keepdims=True)
            a = jnp.exp(ener - emaxn)
            asumn = jnp.sum(mask * asum, axis=1, keepdims=True)
            a = a / (asumn + 1e-16)
            return u + lax.dot_general(
                mask, a * hb, (((0,), (0,)), ((), ())),
                preferred_element_type=_F32)

        u = lax.fori_loop(0, _SNB, sweep3, jnp.zeros((_B, _D), _F32))
        q_star = jnp.concatenate([q, u], axis=1)
        return qh, qc, q_star

    init = (jnp.zeros((_B, _D), _F32), jnp.zeros((_B, _D), _F32),
            jnp.zeros((_B, 2 * _D), _F32))
    _, _, q_star = lax.fori_loop(0, _STEPS, step, init)

    o = jnp.maximum(_bn_apply(_dot(q_star, f1_ref[...]) + g1_ref[...],
                              bnf1_ref[...]), 0.0)
    o = jnp.maximum(_bn_apply(_dot(o, f2_ref[...]) + g2_ref[...],
                              bnf2_ref[...]), 0.0)
    o = jnp.maximum(_bn_apply(_dot(o, f3_ref[...]) + g3_ref[...],
                              bnf3_ref[...]), 0.0)
    out_ref[...] = _dot(o, fo_ref[...]) + go_ref[...]


@jax.jit
def _s2s_head(h, bcol, lwi, lwh, lb, f1, g1, bnf1, f2, g2, bnf2,
              f3, g3, bnf3, fo, go):
    return pl.pallas_call(
        _s2s_body,
        out_shape=jax.ShapeDtypeStruct((_B, 1), _F32),
        scratch_shapes=[pltpu.VMEM((_N, 1), _F32)],
    )(h, bcol, lwi, lwh, lb, f1, g1, bnf1, f2, g2, bnf2, f3, g3, bnf3,
      fo, go)


def kernel(x, edge_attr, params, edge_index, batch):
    p = params

    bn0 = _bn_pack(p["node_bn0_g"], p["node_bn0_b"], p["node_bn0_m"],
                   p["node_bn0_v"])
    bn1 = _bn_pack(p["edge_bn1_g"], p["edge_bn1_b"], p["edge_bn1_m"],
                   p["edge_bn1_v"])
    bnf1 = _bn_pack(p["fc1_bn_g"], p["fc1_bn_b"], p["fc1_bn_m"],
                    p["fc1_bn_v"])
    bnf2 = _bn_pack(p["fc2_bn_g"], p["fc2_bn_b"], p["fc2_bn_m"],
                    p["fc2_bn_v"])
    bnf3 = _bn_pack(p["fc3_bn_g"], p["fc3_bn_b"], p["fc3_bn_m"],
                    p["fc3_bn_v"])

    rmat = jnp.repeat(jnp.eye(_D, dtype=_F32), _D, axis=1)
    fmat = jnp.tile(jnp.eye(_D, dtype=_F32), (_D, 1))

    src = edge_index[0].astype(jnp.int32)
    dst = edge_index[1].astype(jnp.int32)
    emask = _pad_edges(jnp.ones((_E,), jnp.int32)) > 0
    srcp = jnp.where(emask, _pad_edges(src), _N)
    dstp = jnp.where(emask, _pad_edges(dst), _N)
    eap = _pad_edges(edge_attr)
    onesp = _pad_edges(jnp.ones((_E, _D), _F32))

    degp = _sc_scatter(onesp, dstp)
    d0 = degp[0, :_N]
    d1 = degp[1, :_N]

    h, deg = _encoder(x, d0, d1, p["node_lin0_w"],
                      p["node_lin0_b"][None, :], bn0)

    lb = jnp.stack([p["lstm_bi"], p["lstm_bh"]])
    cb = p["conv_bias"][None, :]
    bi = p["gru_bi"][None, :]
    bh = p["gru_bh"][None, :]

    for _ in range(_PASSES):
        hpad = jnp.concatenate([h, jnp.zeros((_NS, _D), _F32)], axis=0)
        xs = _sc_gather(hpad, srcp)
        msg = _msg(eap, xs, p["edge_lin1_w"], p["edge_lin1_b"][None, :],
                   bn1, p["edge_lin2_w"], p["edge_lin2_b"][None, :],
                   rmat, fmat)
        aggp = _sc_scatter(msg, dstp)
        h = _update(aggp[0, :_N], aggp[1, :_N], deg, h, p["conv_root"],
                    cb, p["gru_wi"], p["gru_wh"], bi, bh)

    out = _s2s_head(h, batch.astype(jnp.int32)[:, None], p["lstm_wi"],
                    p["lstm_wh"], lb, p["fc1_w"], p["fc1_b"][None, :], bnf1,
                    p["fc2_w"], p["fc2_b"][None, :], bnf2,
                    p["fc3_w"], p["fc3_b"][None, :], bnf3,
                    p["fco_w"], p["fco_b"][None, :])
    return out.reshape(-1)
